# Initial kernel scaffold; baseline (speedup 1.0000x reference)
import jax
import jax.numpy as jnp
from jax.experimental import pallas as pl


def kernel(x, edge_index, edge_vals, W, b):
    raise NotImplementedError("write your pallas kernel here")


# speedup vs baseline: 4.4400x; 4.4400x over previous
import functools

import jax
import jax.numpy as jnp
from jax import lax
from jax.experimental import pallas as pl
from jax.experimental.pallas import tpu as pltpu
from jax.experimental.pallas import tpu_sc as plsc

_N = 10000
_E = 320000
_D = 128
_H = 128

_NC = 2
_NS = 16
_L = 16

_CH = 80
_EDGES_PER_CORE = _E // _NC
_EDGES_PER_TILE = _EDGES_PER_CORE // _NS
_NCHUNK = _EDGES_PER_TILE // _CH

_ROW_BLK = 80


def _matmul_body(x_ref, w_ref, o_ref):
    o_ref[...] = jnp.dot(x_ref[...], w_ref[...],
                         preferred_element_type=jnp.float32)


def _combine_body(p0_ref, p1_ref, b_ref, o_ref):
    o_ref[...] = p0_ref[...] + p1_ref[...] + b_ref[...]


def _spmm_kernel(h_hbm, src_hbm, dst_hbm, val_hbm, out_hbm,
                 src_v, dst_v, val_v, rows_v, acc_sh, sem):
    c = lax.axis_index("c")
    s = lax.axis_index("s")

    row0 = s * 640
    nblk = jnp.where(s == _NS - 1, 5, 8)

    def _zero_row(r, carry):
        for j in range(_H // _L):
            rows_v[r, pl.ds(j * _L, _L)] = jnp.zeros((_L,), jnp.float32)
        return carry

    lax.fori_loop(0, _CH, _zero_row, 0)

    def _init_blk(t, carry):
        pltpu.sync_copy(rows_v, acc_sh.at[pl.ds(row0 + t * _ROW_BLK, _ROW_BLK)])
        return carry

    lax.fori_loop(0, nblk, _init_blk, 0)
    plsc.subcore_barrier()

    base0 = c * _EDGES_PER_CORE + s * _EDGES_PER_TILE

    def _chunk(k, carry):
        base = base0 + k * _CH
        pltpu.sync_copy(src_hbm.at[pl.ds(base, _CH)], src_v)
        pltpu.sync_copy(dst_hbm.at[pl.ds(base, _CH)], dst_v)
        pltpu.sync_copy(val_hbm.at[pl.ds(base, _CH)], val_v)
        pltpu.async_copy(h_hbm.at[src_v], rows_v, sem).wait()

        def _scale_group(g, carry):
            vals16 = val_v[pl.ds(g * _L, _L)]

            def _one(i, inner):
                idx = (jnp.zeros((_L,), jnp.int32) + i)[:, None]
                ev = lax.gather(
                    vals16, idx,
                    lax.GatherDimensionNumbers(
                        offset_dims=(), collapsed_slice_dims=(0,),
                        start_index_map=(0,)),
                    (1,), mode=lax.GatherScatterMode.PROMISE_IN_BOUNDS)
                e = g * _L + i
                for j in range(_H // _L):
                    sl = (e, pl.ds(j * _L, _L))
                    rows_v[sl] = rows_v[sl] * ev
                return inner

            return lax.fori_loop(0, _L, _one, carry)

        lax.fori_loop(0, _CH // _L, _scale_group, 0)
        pltpu.sync_copy(rows_v, acc_sh.at[dst_v], add=True)
        return carry

    lax.fori_loop(0, _NCHUNK, _chunk, 0)
    plsc.subcore_barrier()

    def _out_blk(t, carry):
        r = row0 + t * _ROW_BLK
        pltpu.sync_copy(acc_sh.at[pl.ds(r, _ROW_BLK)],
                        out_hbm.at[c, pl.ds(r, _ROW_BLK)])
        return carry

    lax.fori_loop(0, nblk, _out_blk, 0)


_spmm = functools.partial(
    pl.kernel,
    mesh=plsc.VectorSubcoreMesh(core_axis_name="c", subcore_axis_name="s"),
    out_type=jax.ShapeDtypeStruct((_NC, _N, _H), jnp.float32),
    scratch_types=[
        pltpu.VMEM((_CH,), jnp.int32),
        pltpu.VMEM((_CH,), jnp.int32),
        pltpu.VMEM((_CH,), jnp.float32),
        pltpu.VMEM((_CH, _H), jnp.float32),
        pltpu.VMEM_SHARED((_N, _H), jnp.float32),
        pltpu.SemaphoreType.DMA,
    ],
)(_spmm_kernel)


def kernel(x, edge_index, edge_vals, W, b):
    bm = 2000
    h = pl.pallas_call(
        _matmul_body,
        out_shape=jax.ShapeDtypeStruct((_N, _H), jnp.float32),
        grid=(_N // bm,),
        in_specs=[
            pl.BlockSpec((bm, _D), lambda i: (i, 0)),
            pl.BlockSpec((_D, _H), lambda i: (0, 0)),
        ],
        out_specs=pl.BlockSpec((bm, _H), lambda i: (i, 0)),
    )(x, W)

    partials = _spmm(h, edge_index[1], edge_index[0], edge_vals)

    out = pl.pallas_call(
        _combine_body,
        out_shape=jax.ShapeDtypeStruct((_N, _H), jnp.float32),
        grid=(_N // bm,),
        in_specs=[
            pl.BlockSpec((bm, _H), lambda i: (i, 0)),
            pl.BlockSpec((bm, _H), lambda i: (i, 0)),
            pl.BlockSpec((1, _H), lambda i: (0, 0)),
        ],
        out_specs=pl.BlockSpec((bm, _H), lambda i: (i, 0)),
    )(partials[0], partials[1], b.reshape(1, _H))
    return out


# speedup vs baseline: 11.8645x; 2.6722x over previous
import functools

import jax
import jax.numpy as jnp
from jax import lax
from jax.experimental import pallas as pl
from jax.experimental.pallas import tpu as pltpu
from jax.experimental.pallas import tpu_sc as plsc

_N = 10000
_E = 320000
_D = 128
_H = 128

_NC = 2
_NS = 16
_NW = _NC * _NS
_L = 16

_CH = 80
_EDGES_PER_TILE = _E // _NW
_NCHUNK = _EDGES_PER_TILE // _CH
_NBUF = 4
_ROW_BLK = 80

_GDNUMS = lax.GatherDimensionNumbers(
    offset_dims=(), collapsed_slice_dims=(0,), start_index_map=(0,))


def _matmul_body(x_ref, w_ref, o_ref):
    o_ref[...] = jnp.dot(x_ref[...], w_ref[...],
                         preferred_element_type=jnp.float32)


def _combine_body(p0_ref, p1_ref, b_ref, o_ref):
    o_ref[...] = p0_ref[...] + p1_ref[...] + b_ref[...]


def _spmm_kernel(h_hbm, src_hbm, dst_hbm, val_hbm, out_hbm, *refs):
    srcb = refs[0:_NBUF]
    dstb = refs[_NBUF:2 * _NBUF]
    valb = refs[2 * _NBUF:3 * _NBUF]
    rows = refs[3 * _NBUF:4 * _NBUF]
    acc_sh = refs[4 * _NBUF]
    esem = refs[4 * _NBUF + 1:4 * _NBUF + 1 + _NBUF]
    gsem = refs[4 * _NBUF + 1 + _NBUF:4 * _NBUF + 1 + 2 * _NBUF]

    c = lax.axis_index("c")
    s = lax.axis_index("s")
    w = c * _NS + s

    def _zero_row(r, carry):
        for j in range(_H // _L):
            rows[0][r, pl.ds(j * _L, _L)] = jnp.zeros((_L,), jnp.float32)
        return carry

    lax.fori_loop(0, _ROW_BLK, _zero_row, 0)
    row0 = s * 640
    nblk = jnp.where(s == _NS - 1, 5, 8)

    def _init_blk(t, carry):
        pltpu.sync_copy(rows[0],
                        acc_sh.at[pl.ds(row0 + t * _ROW_BLK, _ROW_BLK)])
        return carry

    lax.fori_loop(0, nblk, _init_blk, 0)
    plsc.subcore_barrier()

    ebase = w * _EDGES_PER_TILE

    def _start_ed(k, b):
        sl = pl.ds(ebase + k * _CH, _CH)
        pltpu.async_copy(src_hbm.at[sl], srcb[b], esem[b])
        pltpu.async_copy(dst_hbm.at[sl], dstb[b], esem[b])
        pltpu.async_copy(val_hbm.at[sl], valb[b], esem[b])

    def _wait_ed(k, b):
        sl = pl.ds(ebase + k * _CH, _CH)
        pltpu.make_async_copy(src_hbm.at[sl], srcb[b], esem[b]).wait()
        pltpu.make_async_copy(dst_hbm.at[sl], dstb[b], esem[b]).wait()
        pltpu.make_async_copy(val_hbm.at[sl], valb[b], esem[b]).wait()

    def _start_g(k, b):
        pltpu.async_copy(h_hbm.at[srcb[b]], rows[b], gsem[b])

    def _chunk(k, b, do_ed, do_g):
        pltpu.make_async_copy(h_hbm.at[srcb[b]], rows[b], gsem[b]).wait()

        def _scale_group(g, carry):
            vals16 = valb[b][pl.ds(g * _L, _L)]

            def _one(i, inner):
                idx = jnp.zeros((_L,), jnp.int32) + i
                ev = lax.gather(vals16, idx[:, None], _GDNUMS, (1,),
                                mode=lax.GatherScatterMode.PROMISE_IN_BOUNDS)
                e = g * _L + i
                for j in range(_H // _L):
                    sl = (e, pl.ds(j * _L, _L))
                    rows[b][sl] = rows[b][sl] * ev
                return inner

            return lax.fori_loop(0, _L, _one, carry)

        lax.fori_loop(0, _CH // _L, _scale_group, 0)
        pltpu.sync_copy(rows[b], acc_sh.at[dstb[b]], add=True)
        if do_ed:
            _start_ed(k + _NBUF, b)
        if do_g:
            bn = (b + _NBUF - 1) % _NBUF
            _wait_ed(k + _NBUF - 1, bn)
            _start_g(k + _NBUF - 1, bn)

    for b in range(_NBUF):
        _start_ed(b, b)
    for k in range(_NBUF - 1):
        _wait_ed(k, k)
        _start_g(k, k)

    def _outer(o, carry):
        k0 = o * _NBUF
        for b in range(_NBUF):
            _chunk(k0 + b, b, True, True)
        return carry

    n_steady = (_NCHUNK - _NBUF - 1) // _NBUF
    lax.fori_loop(0, n_steady, _outer, 0)
    for k in range(n_steady * _NBUF, _NCHUNK):
        _chunk(k, k % _NBUF,
               do_ed=(k + _NBUF < _NCHUNK),
               do_g=(k + _NBUF - 1 < _NCHUNK))

    plsc.subcore_barrier()

    def _out_blk(t, carry):
        r = row0 + t * _ROW_BLK
        pltpu.sync_copy(acc_sh.at[pl.ds(r, _ROW_BLK)],
                        out_hbm.at[c, pl.ds(r, _ROW_BLK)])
        return carry

    lax.fori_loop(0, nblk, _out_blk, 0)


_spmm = functools.partial(
    pl.kernel,
    mesh=plsc.VectorSubcoreMesh(core_axis_name="c", subcore_axis_name="s"),
    out_type=jax.ShapeDtypeStruct((_NC, _N, _H), jnp.float32),
    scratch_types=(
        [pltpu.VMEM((_CH,), jnp.int32) for _ in range(_NBUF)] +
        [pltpu.VMEM((_CH,), jnp.int32) for _ in range(_NBUF)] +
        [pltpu.VMEM((_CH,), jnp.float32) for _ in range(_NBUF)] +
        [pltpu.VMEM((_CH, _H), jnp.float32) for _ in range(_NBUF)] +
        [pltpu.VMEM_SHARED((_N, _H), jnp.float32)] +
        [pltpu.SemaphoreType.DMA for _ in range(2 * _NBUF)]
    ),
)(_spmm_kernel)


def kernel(x, edge_index, edge_vals, W, b):
    bm = 2000
    h = pl.pallas_call(
        _matmul_body,
        out_shape=jax.ShapeDtypeStruct((_N, _H), jnp.float32),
        grid=(_N // bm,),
        in_specs=[
            pl.BlockSpec((bm, _D), lambda i: (i, 0)),
            pl.BlockSpec((_D, _H), lambda i: (0, 0)),
        ],
        out_specs=pl.BlockSpec((bm, _H), lambda i: (i, 0)),
    )(x, W)

    partials = _spmm(h, edge_index[1], edge_index[0], edge_vals)

    out = pl.pallas_call(
        _combine_body,
        out_shape=jax.ShapeDtypeStruct((_N, _H), jnp.float32),
        grid=(_N // bm,),
        in_specs=[
            pl.BlockSpec((bm, _H), lambda i: (i, 0)),
            pl.BlockSpec((bm, _H), lambda i: (i, 0)),
            pl.BlockSpec((1, _H), lambda i: (0, 0)),
        ],
        out_specs=pl.BlockSpec((bm, _H), lambda i: (i, 0)),
    )(partials[0], partials[1], b.reshape(1, _H))
    return out
